# baseline (speedup 1.0000x reference)
import functools

import jax
import jax.numpy as jnp
from jax import lax
from jax.experimental import pallas as pl
from jax.experimental.pallas import tpu as pltpu
from jax.experimental.pallas import tpu_sc as plsc

N = 256
B = 4
C = 128
TBL = 64
NC, NS = 2, 16
NW = NC * NS
ROWS = N * N * B
RPW = ROWS // NW
CHUNK = 128
NCH = RPW // CHUNK


def _prep_kernel(i_ref, p0_ref, p1_ref, co_ref, table_ref, idx_ref, cm_ref):
    zrow = jnp.zeros((1, C), jnp.float32)
    p0p = jnp.concatenate([p0_ref[...], zrow], axis=0)
    p1p = jnp.concatenate([p1_ref[...], zrow], axis=0)
    table_ref[...] = p0p[:, None, :] + p1p[None, :, :]

    co0 = co_ref[0]
    co1 = co_ref[1]
    for bat in range(B):
        i0 = i_ref[:, bat, 0]
        i1 = i_ref[:, bat, 1]
        r0 = i0[:, None] - i0[None, :] + co0
        r1 = i1[:, None] - i1[None, :] + co1
        cm_ref[bat] = jnp.where((r0 < 0) | (r1 < 0),
                                jnp.int32(1), jnp.int32(0))
        idx_ref[bat] = jnp.maximum(r0, 0) * TBL + jnp.maximum(r1, 0)


def _prep(i, p0, p1, center_offset):
    return pl.pallas_call(
        _prep_kernel,
        in_specs=[
            pl.BlockSpec(memory_space=pltpu.VMEM),
            pl.BlockSpec(memory_space=pltpu.VMEM),
            pl.BlockSpec(memory_space=pltpu.VMEM),
            pl.BlockSpec(memory_space=pltpu.SMEM),
        ],
        out_specs=[
            pl.BlockSpec(memory_space=pltpu.VMEM),
            pl.BlockSpec(memory_space=pltpu.VMEM),
            pl.BlockSpec(memory_space=pltpu.VMEM),
        ],
        out_shape=[
            jax.ShapeDtypeStruct((TBL, TBL, C), jnp.float32),
            jax.ShapeDtypeStruct((B, N, N), jnp.int32),
            jax.ShapeDtypeStruct((B, N, N), jnp.int32),
        ],
    )(i, p0, p1, center_offset)


NBUF = 4


def _gather_body(table_hbm, idx_hbm, out_hbm, idx_v, rows_v, gsem, psem):
    wid = lax.axis_index("s") * NC + lax.axis_index("c")
    pltpu.sync_copy(idx_hbm.at[wid], idx_v)
    base = wid * RPW

    def gather(j, buf):
        return pltpu.make_async_copy(
            table_hbm.at[idx_v.at[j]], rows_v.at[buf], gsem.at[buf])

    def put(j, buf):
        return pltpu.make_async_copy(
            rows_v.at[buf], out_hbm.at[pl.ds(base + j * CHUNK, CHUNK)],
            psem.at[buf])

    for k in range(NBUF - 1):
        gather(k, k).start()

    def body(j, _):
        buf = j % NBUF
        gather(j, buf).wait()
        put(j, buf).start()
        nj = j + NBUF - 1

        @pl.when(nj < NCH)
        def _():
            nbuf = nj % NBUF

            @pl.when(j >= 1)
            def _():
                put(j - 1, nbuf).wait()

            gather(nj, nbuf).start()

        return 0

    lax.fori_loop(0, NCH, body, 0)
    put(NCH - 1, (NCH - 1) % NBUF).wait()


@functools.partial(
    pl.kernel,
    mesh=plsc.VectorSubcoreMesh(core_axis_name="c", subcore_axis_name="s"),
    out_type=jax.ShapeDtypeStruct((ROWS, C), jnp.float32),
    scratch_types=[
        pltpu.VMEM((NCH, CHUNK), jnp.int32),
        pltpu.VMEM((NBUF, CHUNK, C), jnp.float32),
        pltpu.SemaphoreType.DMA((NBUF,)),
        pltpu.SemaphoreType.DMA((NBUF,)),
    ],
)
def _sc_gather(table_hbm, idx_hbm, out_hbm, idx_v, rows_v, gsem, psem):
    _gather_body(table_hbm, idx_hbm, out_hbm, idx_v, rows_v, gsem, psem)


def kernel(i, p0, p1, center_offset):
    i = i.astype(jnp.int32)
    center_offset = center_offset.astype(jnp.int32)
    table, idx, cm = _prep(i, p0, p1, center_offset)
    table = table.reshape(TBL * TBL, C)
    idx = jnp.transpose(idx, (1, 2, 0)).reshape(NW, NCH, CHUNK)
    pe = _sc_gather(table, idx)
    pe = pe.reshape(N, N, B, C)
    cm = jnp.transpose(cm, (1, 2, 0)).astype(jnp.bool_)
    return pe, cm


# speedup vs baseline: 1.3027x; 1.3027x over previous
import functools

import jax
import jax.numpy as jnp
from jax import lax
from jax.experimental import pallas as pl
from jax.experimental.pallas import tpu as pltpu
from jax.experimental.pallas import tpu_sc as plsc

N = 256
B = 4
C = 128
TBL = 64
NC, NS = 2, 16
NW = NC * NS
ROWS = N * N * B
RPW = ROWS // NW
CHUNK = 128
NCH = RPW // CHUNK


def _prep_kernel(i0_ref, i1_ref, i0f_ref, i1f_ref, p0_ref, p1_ref, co_ref,
                 table_ref, idx_ref, cm_ref):
    zrow = jnp.zeros((1, C), jnp.float32)
    p0p = jnp.concatenate([p0_ref[...], zrow], axis=0)
    p1p = jnp.concatenate([p1_ref[...], zrow], axis=0)
    table_ref[...] = p0p[:, None, :] + p1p[None, :, :]

    co0 = co_ref[0]
    co1 = co_ref[1]
    q = lax.broadcasted_iota(jnp.int32, (N, N * B), 1)
    bat = q & (B - 1)

    def xterm(ref):
        a = jnp.broadcast_to(ref[:, 0:1], (N, N * B))
        for k in range(1, B):
            a = jnp.where(bat == k, ref[:, k:k + 1], a)
        return a

    a0 = xterm(i0_ref)
    a1 = xterm(i1_ref)
    r0 = a0 - i0f_ref[...] + co0
    r1 = a1 - i1f_ref[...] + co1
    cm_ref[...] = jnp.where((r0 < 0) | (r1 < 0), jnp.int32(1), jnp.int32(0))
    idx_ref[...] = jnp.maximum(r0, 0) * TBL + jnp.maximum(r1, 0)


def _prep(i0, i1, i0f, i1f, p0, p1, center_offset):
    return pl.pallas_call(
        _prep_kernel,
        in_specs=[pl.BlockSpec(memory_space=pltpu.VMEM)] * 6 +
                 [pl.BlockSpec(memory_space=pltpu.SMEM)],
        out_specs=[
            pl.BlockSpec(memory_space=pltpu.VMEM),
            pl.BlockSpec(memory_space=pltpu.VMEM),
            pl.BlockSpec(memory_space=pltpu.VMEM),
        ],
        out_shape=[
            jax.ShapeDtypeStruct((TBL, TBL, C), jnp.float32),
            jax.ShapeDtypeStruct((N, N * B), jnp.int32),
            jax.ShapeDtypeStruct((N, N * B), jnp.int32),
        ],
    )(i0, i1, i0f, i1f, p0, p1, center_offset)


NBUF = 4


def _gather_body(table_hbm, idx_hbm, out_hbm, idx_v, rows_v, gsem, psem):
    wid = lax.axis_index("s") * NC + lax.axis_index("c")
    pltpu.sync_copy(idx_hbm.at[wid], idx_v)
    base = wid * RPW

    def gather(j, buf):
        return pltpu.make_async_copy(
            table_hbm.at[idx_v.at[j]], rows_v.at[buf], gsem.at[buf])

    def put(j, buf):
        return pltpu.make_async_copy(
            rows_v.at[buf], out_hbm.at[pl.ds(base + j * CHUNK, CHUNK)],
            psem.at[buf])

    for k in range(NBUF - 1):
        gather(k, k).start()

    def body(j, _):
        buf = j % NBUF
        gather(j, buf).wait()
        put(j, buf).start()
        nj = j + NBUF - 1

        @pl.when(nj < NCH)
        def _():
            nbuf = nj % NBUF

            @pl.when(j >= 1)
            def _():
                put(j - 1, nbuf).wait()

            gather(nj, nbuf).start()

        return 0

    lax.fori_loop(0, NCH, body, 0)
    put(NCH - 1, (NCH - 1) % NBUF).wait()


@functools.partial(
    pl.kernel,
    mesh=plsc.VectorSubcoreMesh(core_axis_name="c", subcore_axis_name="s"),
    out_type=jax.ShapeDtypeStruct((ROWS, C), jnp.float32),
    scratch_types=[
        pltpu.VMEM((NCH, CHUNK), jnp.int32),
        pltpu.VMEM((NBUF, CHUNK, C), jnp.float32),
        pltpu.SemaphoreType.DMA((NBUF,)),
        pltpu.SemaphoreType.DMA((NBUF,)),
    ],
)
def _sc_gather(table_hbm, idx_hbm, out_hbm, idx_v, rows_v, gsem, psem):
    _gather_body(table_hbm, idx_hbm, out_hbm, idx_v, rows_v, gsem, psem)


def kernel(i, p0, p1, center_offset):
    i = i.astype(jnp.int32)
    center_offset = center_offset.astype(jnp.int32)
    i0 = i[:, :, 0]
    i1 = i[:, :, 1]
    i0f = i0.reshape(1, N * B)
    i1f = i1.reshape(1, N * B)
    table, idx, cm = _prep(i0, i1, i0f, i1f, p0, p1, center_offset)
    table = table.reshape(TBL * TBL, C)
    idx = idx.reshape(NW, NCH, CHUNK)
    pe = _sc_gather(table, idx)
    pe = pe.reshape(N, N, B, C)
    cm = cm.reshape(N, N, B).astype(jnp.bool_)
    return pe, cm


# speedup vs baseline: 3.2487x; 2.4938x over previous
import functools

import jax
import jax.numpy as jnp
from jax import lax
from jax.experimental import pallas as pl
from jax.experimental.pallas import tpu as pltpu
from jax.experimental.pallas import tpu_sc as plsc

N = 256
B = 4
C = 128
TBL = 64
NC, NS = 2, 16
NW = NC * NS
ROWS = N * N * B
RPW = ROWS // NW
CHUNK = 128
NCH = RPW // CHUNK


def _prep_kernel(i0_ref, i1_ref, i0f_ref, i1f_ref, p0_ref, p1_ref, co_ref,
                 table_ref, idx_ref, cm_ref):
    zrow = jnp.zeros((1, C), jnp.float32)
    p0p = jnp.concatenate([p0_ref[...], zrow], axis=0)
    p1p = jnp.concatenate([p1_ref[...], zrow], axis=0)
    table_ref[...] = p0p[:, None, :] + p1p[None, :, :]

    co0 = co_ref[0]
    co1 = co_ref[1]
    q = lax.broadcasted_iota(jnp.int32, (N, N * B), 1)
    bat = q & (B - 1)

    def xterm(ref):
        a = jnp.broadcast_to(ref[:, 0:1], (N, N * B))
        for k in range(1, B):
            a = jnp.where(bat == k, ref[:, k:k + 1], a)
        return a

    a0 = xterm(i0_ref)
    a1 = xterm(i1_ref)
    r0 = a0 - i0f_ref[...] + co0
    r1 = a1 - i1f_ref[...] + co1
    cm_ref[...] = jnp.where((r0 < 0) | (r1 < 0), jnp.int32(1), jnp.int32(0))
    idx_ref[...] = jnp.maximum(r0, 0) * TBL + jnp.maximum(r1, 0)


def _prep(i0, i1, i0f, i1f, p0, p1, center_offset):
    return pl.pallas_call(
        _prep_kernel,
        in_specs=[pl.BlockSpec(memory_space=pltpu.VMEM)] * 6 +
                 [pl.BlockSpec(memory_space=pltpu.SMEM)],
        out_specs=[
            pl.BlockSpec(memory_space=pltpu.VMEM),
            pl.BlockSpec(memory_space=pltpu.VMEM),
            pl.BlockSpec(memory_space=pltpu.VMEM),
        ],
        out_shape=[
            jax.ShapeDtypeStruct((TBL, TBL, C), jnp.float32),
            jax.ShapeDtypeStruct((N, N * B), jnp.int32),
            jax.ShapeDtypeStruct((N, N * B), jnp.int32),
        ],
    )(i0, i1, i0f, i1f, p0, p1, center_offset)


NBUF = 4


def _gather_body(table_hbm, idx_hbm, out_hbm, idx_v, rows_v, tbl_sh, gsem, psem):
    sid = lax.axis_index("s")
    wid = sid * NC + lax.axis_index("c")
    @pl.when(sid == 0)
    def _():
        pltpu.sync_copy(table_hbm, tbl_sh)
    pltpu.sync_copy(idx_hbm.at[wid], idx_v)
    plsc.subcore_barrier()
    base = wid * RPW

    def gather(j, buf):
        return pltpu.make_async_copy(
            tbl_sh.at[idx_v.at[j]], rows_v.at[buf], gsem.at[buf])

    def put(j, buf):
        return pltpu.make_async_copy(
            rows_v.at[buf], out_hbm.at[pl.ds(base + j * CHUNK, CHUNK)],
            psem.at[buf])

    for k in range(NBUF - 1):
        gather(k, k).start()

    def body(j, _):
        buf = j % NBUF
        gather(j, buf).wait()
        put(j, buf).start()
        nj = j + NBUF - 1

        @pl.when(nj < NCH)
        def _():
            nbuf = nj % NBUF

            @pl.when(j >= 1)
            def _():
                put(j - 1, nbuf).wait()

            gather(nj, nbuf).start()

        return 0

    lax.fori_loop(0, NCH, body, 0)
    put(NCH - 1, (NCH - 1) % NBUF).wait()


@functools.partial(
    pl.kernel,
    mesh=plsc.VectorSubcoreMesh(core_axis_name="c", subcore_axis_name="s"),
    out_type=jax.ShapeDtypeStruct((ROWS, C), jnp.float32),
    scratch_types=[
        pltpu.VMEM((NCH, CHUNK), jnp.int32),
        pltpu.VMEM((NBUF, CHUNK, C), jnp.float32),
        pltpu.VMEM_SHARED((TBL * TBL, C), jnp.float32),
        pltpu.SemaphoreType.DMA((NBUF,)),
        pltpu.SemaphoreType.DMA((NBUF,)),
    ],
)
def _sc_gather(table_hbm, idx_hbm, out_hbm, idx_v, rows_v, tbl_sh, gsem, psem):
    _gather_body(table_hbm, idx_hbm, out_hbm, idx_v, rows_v, tbl_sh, gsem, psem)


def kernel(i, p0, p1, center_offset):
    i = i.astype(jnp.int32)
    center_offset = center_offset.astype(jnp.int32)
    i0 = i[:, :, 0]
    i1 = i[:, :, 1]
    i0f = i0.reshape(1, N * B)
    i1f = i1.reshape(1, N * B)
    table, idx, cm = _prep(i0, i1, i0f, i1f, p0, p1, center_offset)
    table = table.reshape(TBL * TBL, C)
    idx = idx.reshape(NW, NCH, CHUNK)
    pe = _sc_gather(table, idx)
    pe = pe.reshape(N, N, B, C)
    cm = cm.reshape(N, N, B).astype(jnp.bool_)
    return pe, cm
